# baseline (speedup 1.0000x reference)
import functools

import jax
import jax.numpy as jnp
from jax.experimental import pallas as pl
from jax.experimental.pallas import tpu as pltpu

_BM = 512
_BK = 2048


def _stage_body(nk, scale, a_ref, x_ref, r_ref, o_ref):
    k = pl.program_id(1)

    @pl.when(k == 0)
    def _():
        o_ref[...] = jnp.zeros_like(o_ref)

    o_ref[...] += jnp.dot(a_ref[...], x_ref[...],
                          preferred_element_type=jnp.float32)

    @pl.when(k == nk - 1)
    def _():
        o_ref[...] = (o_ref[...] + r_ref[...]) * scale


def _stage(a, x, resid, scale):
    m, kdim = a.shape
    n = x.shape[1]
    nk = kdim // _BK
    grid = (m // _BM, nk)
    return pl.pallas_call(
        functools.partial(_stage_body, nk, scale),
        grid=grid,
        in_specs=[
            pl.BlockSpec((_BM, _BK), lambda i, k: (i, k)),
            pl.BlockSpec((_BK, n), lambda i, k: (k, 0)),
            pl.BlockSpec((_BM, n), lambda i, k: (i, 0)),
        ],
        out_specs=pl.BlockSpec((_BM, n), lambda i, k: (i, 0)),
        out_shape=jax.ShapeDtypeStruct((m, n), jnp.float32),
        compiler_params=pltpu.CompilerParams(
            dimension_semantics=("parallel", "arbitrary"),
        ),
    )(a, x, resid)


def kernel(pois_embs, HG_up, HG_pu):
    p0 = pois_embs
    zeros_u = jnp.zeros((HG_up.shape[0], p0.shape[1]), jnp.float32)
    m1 = _stage(HG_up, p0, zeros_u, 1.0)
    p1 = _stage(HG_pu, m1, p0, 1.0)
    m2 = _stage(HG_up, p1, zeros_u, 1.0)
    out = _stage(HG_pu, m2, p0 + 2.0 * p1, 1.0 / 3.0)
    return out
